# baseline (speedup 1.0000x reference)
import functools

import jax
import jax.numpy as jnp
from jax import lax
from jax.experimental import pallas as pl
from jax.experimental.pallas import tpu as pltpu
from jax.experimental.pallas import tpu_sc as plsc

NUM_EMB = 1000000
DIM = 32
N_LOOKUPS = 327680
LANES = 16
NS = 16

TILE_COLS = NUM_EMB // 128
MAIN_ROWS = TILE_COLS * 128
TAIL = NUM_EMB - MAIN_ROWS

COLS_PER_TILE = 488 * 128
RUN = 512
N_RUNS = COLS_PER_TILE // RUN
EXTRA_BASE = 16 * COLS_PER_TILE

CH = 256
B_PER_TILE = N_LOOKUPS // NS
N_CHUNKS = B_PER_TILE // CH

_IOTA = lambda: lax.iota(jnp.int32, LANES)


def _sc_body(idx0, idx1, t0T, t1T, tail0, tail1,
             embT0, embT1, parts, conv0, conv1,
             bufw, st0, st1, idx_all, tailv, accv,
             sem_r, sem_w0, sem_w1, sem_o):
    c = lax.axis_index("c")
    s = lax.axis_index("s")
    sts = (st0, st1)
    sems_w = (sem_w0, sem_w1)

    def scatter_block(src_col0, st, n_rows):
        zeros = jnp.zeros((LANES,), jnp.int32)

        def g_body(g, _):
            rows = g * LANES + _IOTA()
            for d in range(DIM):
                v = bufw[d, pl.ds(src_col0 + g * LANES, LANES)]
                plsc.store_scatter(st, [rows, zeros, jnp.full((LANES,), d, jnp.int32)], v)
            return 0

        lax.fori_loop(0, n_rows // LANES, g_body, 0)

    def convert(tT, tail, conv):
        j_tile = s * COLS_PER_TILE

        def run_body(r, _):
            j0 = j_tile + r * RUN
            pltpu.sync_copy(tT.at[pl.ds(0, DIM), pl.ds(pl.multiple_of(j0, 128), RUN)], bufw)
            for h in range(2):
                scatter_block(h * CH, sts[h], CH)
                pltpu.async_copy(
                    sts[h], conv.at[pl.ds(pl.multiple_of(j0 + h * CH, 8), CH)], sems_w[h])
                pltpu.make_async_copy(
                    sts[h], conv.at[pl.ds(0, CH)], sems_w[h]).wait()
            return 0

        lax.fori_loop(0, N_RUNS, run_body, 0)

        @pl.when(s < 4)
        def _extra():
            j0 = EXTRA_BASE + s * 128
            pltpu.sync_copy(tT.at[pl.ds(0, DIM), pl.ds(pl.multiple_of(j0, 128), 128)],
                            bufw.at[pl.ds(0, DIM), pl.ds(0, 128)])
            scatter_block(0, st0, 128)
            pltpu.sync_copy(st0.at[pl.ds(0, 128)],
                            conv.at[pl.ds(pl.multiple_of(j0, 8), 128)])

        @pl.when(s == 4)
        def _tail():
            pltpu.sync_copy(tail, tailv)

            def t_body(r, _):
                for h in range(2):
                    st0[r, 0, pl.ds(h * LANES, LANES)] = tailv[pl.ds(r * DIM + h * LANES, LANES)]
                return 0

            lax.fori_loop(0, TAIL, t_body, 0)
            pltpu.sync_copy(st0.at[pl.ds(0, TAIL)], conv.at[pl.ds(MAIN_ROWS, TAIL)])

    def gather(idx, conv, embT):
        base = s * B_PER_TILE
        rows_per_tile = B_PER_TILE // 128
        pltpu.sync_copy(idx.at[pl.ds(s * rows_per_tile, rows_per_tile)], idx_all)

        def chunk_body(i, carry):
            a, b = carry
            for q in range(CH // 128):
                pltpu.async_copy(conv.at[idx_all.at[i * (CH // 128) + q]],
                                 st0.at[pl.ds(q * 128, 128)], sem_r)
            for q in range(CH // 128):
                pltpu.make_async_copy(conv.at[idx_all.at[0]],
                                      st0.at[pl.ds(0, 128)], sem_r).wait()

            def row_body(j, carry2):
                a2, b2 = carry2
                v0 = st0[j, 0, pl.ds(0, LANES)]
                v1 = st0[j, 0, pl.ds(LANES, LANES)]
                return (a2 + v0, b2 + v1)

            a, b = lax.fori_loop(0, CH, row_body, (a, b))
            off = base + i * CH
            pltpu.sync_copy(st0, embT.at[pl.ds(pl.multiple_of(off, CH), CH)])
            return (a, b)

        zero = jnp.zeros((LANES,), jnp.float32)
        a, b = lax.fori_loop(0, N_CHUNKS, chunk_body, (zero, zero))
        accv[...] = a + b
        wid = c * NS + s
        pltpu.sync_copy(accv, parts.at[pl.ds(wid * LANES, LANES)])

    @pl.when(c == 0)
    def _table0():
        convert(t0T, tail0, conv0)

    @pl.when(c == 1)
    def _table1():
        convert(t1T, tail1, conv1)

    plsc.subcore_barrier()

    @pl.when(c == 0)
    def _gather0():
        gather(idx0, conv0, embT0)

    @pl.when(c == 1)
    def _gather1():
        gather(idx1, conv1, embT1)


@jax.jit
def kernel(indices_0, indices_1, table_0, table_1):
    mesh = plsc.VectorSubcoreMesh(core_axis_name="c", subcore_axis_name="s")
    call = functools.partial(
        pl.kernel,
        mesh=mesh,
        compiler_params=pltpu.CompilerParams(
            use_tc_tiling_on_sc=True, needs_layout_passes=False),
        out_type=(
            jax.ShapeDtypeStruct((N_LOOKUPS, 1, DIM), jnp.float32),
            jax.ShapeDtypeStruct((N_LOOKUPS, 1, DIM), jnp.float32),
            jax.ShapeDtypeStruct((2 * NS * LANES,), jnp.float32),
            jax.ShapeDtypeStruct((NUM_EMB, 1, DIM), jnp.float32),
            jax.ShapeDtypeStruct((NUM_EMB, 1, DIM), jnp.float32),
        ),
        scratch_types=[
            pltpu.VMEM((DIM, RUN), jnp.float32),
            pltpu.VMEM((CH, 1, DIM), jnp.float32),
            pltpu.VMEM((CH, 1, DIM), jnp.float32),
            pltpu.VMEM((B_PER_TILE // 128, 128), jnp.int32),
            pltpu.VMEM((TAIL * DIM,), jnp.float32),
            pltpu.VMEM((LANES,), jnp.float32),
            pltpu.SemaphoreType.DMA,
            pltpu.SemaphoreType.DMA,
            pltpu.SemaphoreType.DMA,
            pltpu.SemaphoreType.DMA,
        ],
    )(_sc_body)
    tail0 = table_0[MAIN_ROWS:].reshape(-1)
    tail1 = table_1[MAIN_ROWS:].reshape(-1)
    emb3_0, emb3_1, parts, _, _ = call(
        indices_0.reshape(-1, 128), indices_1.reshape(-1, 128),
        table_0.T, table_1.T, tail0, tail1)
    loss = jnp.sum(parts) / jnp.float32(2 * N_LOOKUPS * DIM)
    return (loss, emb3_0[:, 0, :], emb3_1[:, 0, :])


# speedup vs baseline: 1.6823x; 1.6823x over previous
import functools

import jax
import jax.numpy as jnp
from jax import lax
from jax.experimental import pallas as pl
from jax.experimental.pallas import tpu as pltpu
from jax.experimental.pallas import tpu_sc as plsc

NUM_EMB = 1000000
DIM = 32
N_LOOKUPS = 327680

_INFO = plsc.get_sparse_core_info()
NC = _INFO.num_cores
NS = _INFO.num_subcores
NW = NC * NS
LANES = _INFO.num_lanes

B_PER_W = N_LOOKUPS // NW
CHUNK = 2048
N_CHUNKS = B_PER_W // CHUNK


def _sc_lookup(idx0_hbm, idx1_hbm, t0_hbm, t1_hbm,
               out0_hbm, out1_hbm, parts_hbm,
               idx_v, rows_v, acc_v, sem):
    wid = lax.axis_index("s") * NC + lax.axis_index("c")
    base = wid * B_PER_W

    def do_table(idx_hbm, t_hbm, out_hbm, carry):
        def chunk_body(i, carry):
            off = base + i * CHUNK
            pltpu.sync_copy(idx_hbm.at[pl.ds(off, CHUNK)], idx_v)
            pltpu.async_copy(t_hbm.at[idx_v], rows_v, sem).wait()
            pltpu.sync_copy(rows_v, out_hbm.at[pl.ds(off, CHUNK)])

            def sum_body(j, c):
                a, b = c
                return (a + rows_v[j, pl.ds(0, LANES)],
                        b + rows_v[j, pl.ds(LANES, LANES)])

            return lax.fori_loop(0, CHUNK, sum_body, carry, unroll=8)

        return lax.fori_loop(0, N_CHUNKS, chunk_body, carry)

    zeros = jnp.zeros((LANES,), jnp.float32)
    carry = (zeros, zeros)
    carry = do_table(idx0_hbm, t0_hbm, out0_hbm, carry)
    carry = do_table(idx1_hbm, t1_hbm, out1_hbm, carry)
    acc_v[...] = carry[0] + carry[1]
    pltpu.sync_copy(acc_v, parts_hbm.at[wid])


@jax.jit
def kernel(indices_0, indices_1, table_0, table_1):
    mesh = plsc.VectorSubcoreMesh(core_axis_name="c", subcore_axis_name="s")
    call = functools.partial(
        pl.kernel,
        mesh=mesh,
        compiler_params=pltpu.CompilerParams(use_tc_tiling_on_sc=False),
        out_type=(
            jax.ShapeDtypeStruct((N_LOOKUPS, DIM), jnp.float32),
            jax.ShapeDtypeStruct((N_LOOKUPS, DIM), jnp.float32),
            jax.ShapeDtypeStruct((NW, LANES), jnp.float32),
        ),
        scratch_types=[
            pltpu.VMEM((CHUNK,), jnp.int32),
            pltpu.VMEM((CHUNK, DIM), jnp.float32),
            pltpu.VMEM((LANES,), jnp.float32),
            pltpu.SemaphoreType.DMA,
        ],
    )(_sc_lookup)
    t0 = table_0.reshape(-1).reshape(NUM_EMB, DIM)
    t1 = table_1.reshape(-1).reshape(NUM_EMB, DIM)
    emb_0, emb_1, parts = call(indices_0, indices_1, t0, t1)
    loss = jnp.sum(parts) / jnp.float32(2 * N_LOOKUPS * DIM)
    return (loss, emb_0, emb_1)
